# baseline (speedup 1.0000x reference)
import jax
import jax.numpy as jnp
from jax.experimental import pallas as pl
from jax.experimental.pallas import tpu as pltpu

_B, _N, _D = 32, 2048, 1024
_TOPK = 25
_NB = 512
_NBLK = _N // _NB


def _score_body(v_ref, aud_ref, nf_ref, na_ref, s1_ref, s2_ref):
    x1 = nf_ref[0]
    x2 = na_ref[0]
    vb = v_ref[0]
    ab = aud_ref[0]
    ones = jnp.ones((1, _D), jnp.float32)
    dn = (((1,), (1,)), ((), ()))
    hi = jax.lax.Precision.HIGHEST
    d1 = jax.lax.dot_general(vb, x1, dn, precision=hi, preferred_element_type=jnp.float32)
    d2 = jax.lax.dot_general(ab, x2, dn, precision=hi, preferred_element_type=jnp.float32)
    ss1 = jax.lax.dot_general(ones, x1 * x1, dn, precision=hi, preferred_element_type=jnp.float32)
    ss2 = jax.lax.dot_general(ones, x2 * x2, dn, precision=hi, preferred_element_type=jnp.float32)
    nv = jnp.sqrt(jnp.sum(vb * vb))
    nu = jnp.sqrt(jnp.sum(ab * ab))
    n1 = jnp.sqrt(ss1)
    n2 = jnp.sqrt(ss2)
    nv_c = jnp.maximum(nv, 1e-12)
    nu_c = jnp.maximum(nu, 1e-12)
    n1_c = jnp.maximum(n1, 1e-12)
    n2_c = jnp.maximum(n2, 1e-12)
    c1 = (d1 / (nv_c * n1_c)) / jnp.maximum((nv / nv_c) * (n1 / n1_c), 1e-8)
    c2 = (d2 / (nu_c * n2_c)) / jnp.maximum((nu / nu_c) * (n2 / n2_c), 1e-8)
    s1_ref[0] = c1
    s2_ref[0] = c2


def _mask_body(temp_ref, ids_ref, s1_ref, s2_ref, w_ref, cur_ref, rem_ref):
    t = temp_ref[0, 0]
    a = 1.0 / (1.0 + jnp.exp(-t))
    mask = (ids_ref[:, 0, :] != -1).astype(jnp.float32)
    c1 = jnp.clip(s1_ref[:, 0, :] * mask, 0.0, 1.0)
    c2 = jnp.clip(s2_ref[:, 0, :] * mask, 0.0, 1.0)
    e1 = jnp.exp(c1 - jnp.max(c1, axis=-1, keepdims=True))
    sm1 = e1 / jnp.sum(e1, axis=-1, keepdims=True)
    e2 = jnp.exp(c2 - jnp.max(c2, axis=-1, keepdims=True))
    sm2 = e2 / jnp.sum(e2, axis=-1, keepdims=True)
    att = a * sm1 + (1.0 - a) * sm2
    iota = jax.lax.broadcasted_iota(jnp.int32, (_B, _N), 1)

    cur_ref[...] = att
    rem_ref[...] = jnp.zeros((_B, _N), jnp.float32)

    def step(_, c):
        cur = cur_ref[...]
        mx = jnp.max(cur, axis=-1, keepdims=True)
        cand = jnp.where(cur == mx, iota, _N)
        first = jnp.min(cand, axis=-1, keepdims=True)
        hit = iota == first
        cur_ref[...] = jnp.where(hit, -1.0, cur)
        rem_ref[...] = jnp.maximum(rem_ref[...], hit.astype(jnp.float32))
        return c

    jax.lax.fori_loop(0, _TOPK, step, 0)
    m = jnp.max(rem_ref[...], axis=0, keepdims=True)
    w_ref[:, 0, :] = att * m


def _aggr_body(w_ref, x_ref, out_ref):
    j = pl.program_id(1)
    part = jax.lax.dot_general(w_ref[0], x_ref[0], (((1,), (0,)), ((), ())),
                               precision=jax.lax.Precision.HIGHEST,
                               preferred_element_type=jnp.float32)

    @pl.when(j == 0)
    def _():
        out_ref[0] = part

    @pl.when(j > 0)
    def _():
        out_ref[0] += part


def _final_body(aggr_ref, o_ref, out_ref):
    out_ref[...] = jnp.sum(aggr_ref[:, 0, :][:, None, :] * o_ref[...], axis=-1)


def kernel(v, aud, o, n_feats, n_ids, n_answ, n_auds, temp):
    f32 = jnp.float32
    v3 = v.reshape(_B, 1, _D)
    aud3 = aud.reshape(_B, 1, _D)
    s1, s2 = pl.pallas_call(
        _score_body,
        grid=(_B, _NBLK),
        in_specs=[
            pl.BlockSpec((1, 1, _D), lambda b, j: (b, 0, 0)),
            pl.BlockSpec((1, 1, _D), lambda b, j: (b, 0, 0)),
            pl.BlockSpec((1, _NB, _D), lambda b, j: (b, j, 0)),
            pl.BlockSpec((1, _NB, _D), lambda b, j: (b, j, 0)),
        ],
        out_specs=[
            pl.BlockSpec((1, 1, _NB), lambda b, j: (b, 0, j)),
            pl.BlockSpec((1, 1, _NB), lambda b, j: (b, 0, j)),
        ],
        out_shape=[jax.ShapeDtypeStruct((_B, 1, _N), f32)] * 2,
        compiler_params=pltpu.CompilerParams(
            dimension_semantics=("parallel", "parallel")),
    )(v3, aud3, n_feats, n_auds)

    w = pl.pallas_call(
        _mask_body,
        in_specs=[
            pl.BlockSpec((1, 1), lambda: (0, 0)),
            pl.BlockSpec((_B, 1, _N), lambda: (0, 0, 0)),
            pl.BlockSpec((_B, 1, _N), lambda: (0, 0, 0)),
            pl.BlockSpec((_B, 1, _N), lambda: (0, 0, 0)),
        ],
        out_specs=pl.BlockSpec((_B, 1, _N), lambda: (0, 0, 0)),
        out_shape=jax.ShapeDtypeStruct((_B, 1, _N), f32),
        scratch_shapes=[pltpu.VMEM((_B, _N), f32), pltpu.VMEM((_B, _N), f32)],
    )(temp.reshape(1, 1), n_ids, s1, s2)

    aggr = pl.pallas_call(
        _aggr_body,
        grid=(_B, _NBLK),
        in_specs=[
            pl.BlockSpec((1, 1, _NB), lambda b, j: (b, 0, j)),
            pl.BlockSpec((1, _NB, _D), lambda b, j: (b, j, 0)),
        ],
        out_specs=pl.BlockSpec((1, 1, _D), lambda b, j: (b, 0, 0)),
        out_shape=jax.ShapeDtypeStruct((_B, 1, _D), f32),
        compiler_params=pltpu.CompilerParams(
            dimension_semantics=("parallel", "arbitrary")),
    )(w, n_answ)

    scores = pl.pallas_call(
        _final_body,
        in_specs=[
            pl.BlockSpec((_B, 1, _D), lambda: (0, 0, 0)),
            pl.BlockSpec((_B, 3, _D), lambda: (0, 0, 0)),
        ],
        out_specs=pl.BlockSpec((_B, 3), lambda: (0, 0)),
        out_shape=jax.ShapeDtypeStruct((_B, 3), f32),
    )(aggr, o)
    return scores


# speedup vs baseline: 2.8215x; 2.8215x over previous
import jax
import jax.numpy as jnp
from jax.experimental import pallas as pl
from jax.experimental.pallas import tpu as pltpu

_B, _N, _D = 32, 2048, 1024
_TOPK = 25
_NB = 512
_NBLK = _N // _NB


def _score_body(v_ref, aud_ref, nf_ref, na_ref, s1_ref, s2_ref):
    x1 = nf_ref[0]
    x2 = na_ref[0]
    vb = v_ref[0]
    ab = aud_ref[0]
    d1 = jnp.sum(x1 * vb, axis=1).reshape(1, _NB)
    d2 = jnp.sum(x2 * ab, axis=1).reshape(1, _NB)
    ss1 = jnp.sum(x1 * x1, axis=1).reshape(1, _NB)
    ss2 = jnp.sum(x2 * x2, axis=1).reshape(1, _NB)
    nv = jnp.sqrt(jnp.sum(vb * vb))
    nu = jnp.sqrt(jnp.sum(ab * ab))
    n1 = jnp.sqrt(ss1)
    n2 = jnp.sqrt(ss2)
    nv_c = jnp.maximum(nv, 1e-12)
    nu_c = jnp.maximum(nu, 1e-12)
    n1_c = jnp.maximum(n1, 1e-12)
    n2_c = jnp.maximum(n2, 1e-12)
    c1 = (d1 / (nv_c * n1_c)) / jnp.maximum((nv / nv_c) * (n1 / n1_c), 1e-8)
    c2 = (d2 / (nu_c * n2_c)) / jnp.maximum((nu / nu_c) * (n2 / n2_c), 1e-8)
    s1_ref[0] = c1
    s2_ref[0] = c2


def _mask_body(temp_ref, ids_ref, s1_ref, s2_ref, w_ref, cur_ref, rem_ref):
    t = temp_ref[0, 0]
    a = 1.0 / (1.0 + jnp.exp(-t))
    mask = (ids_ref[:, 0, :] != -1).astype(jnp.float32)
    c1 = jnp.clip(s1_ref[:, 0, :] * mask, 0.0, 1.0)
    c2 = jnp.clip(s2_ref[:, 0, :] * mask, 0.0, 1.0)
    e1 = jnp.exp(c1 - jnp.max(c1, axis=-1, keepdims=True))
    sm1 = e1 / jnp.sum(e1, axis=-1, keepdims=True)
    e2 = jnp.exp(c2 - jnp.max(c2, axis=-1, keepdims=True))
    sm2 = e2 / jnp.sum(e2, axis=-1, keepdims=True)
    att = a * sm1 + (1.0 - a) * sm2
    iota = jax.lax.broadcasted_iota(jnp.int32, (_B, _N), 1)

    cur_ref[...] = att
    rem_ref[...] = jnp.zeros((_B, _N), jnp.float32)

    def step(_, c):
        cur = cur_ref[...]
        mx = jnp.max(cur, axis=-1, keepdims=True)
        cand = jnp.where(cur == mx, iota, _N)
        first = jnp.min(cand, axis=-1, keepdims=True)
        hit = iota == first
        cur_ref[...] = jnp.where(hit, -1.0, cur)
        rem_ref[...] = jnp.maximum(rem_ref[...], hit.astype(jnp.float32))
        return c

    jax.lax.fori_loop(0, _TOPK, step, 0)
    m = jnp.max(rem_ref[...], axis=0, keepdims=True)
    w_ref[:, 0, :] = att * m


def _aggr_body(w_ref, x_ref, out_ref):
    j = pl.program_id(1)
    part = jax.lax.dot_general(w_ref[0], x_ref[0], (((1,), (0,)), ((), ())),
                               preferred_element_type=jnp.float32)

    @pl.when(j == 0)
    def _():
        out_ref[0] = part

    @pl.when(j > 0)
    def _():
        out_ref[0] += part


def _final_body(aggr_ref, o_ref, out_ref):
    out_ref[...] = jnp.sum(aggr_ref[:, 0, :][:, None, :] * o_ref[...], axis=-1)


def kernel(v, aud, o, n_feats, n_ids, n_answ, n_auds, temp):
    f32 = jnp.float32
    v3 = v.reshape(_B, 1, _D)
    aud3 = aud.reshape(_B, 1, _D)
    s1, s2 = pl.pallas_call(
        _score_body,
        grid=(_B, _NBLK),
        in_specs=[
            pl.BlockSpec((1, 1, _D), lambda b, j: (b, 0, 0)),
            pl.BlockSpec((1, 1, _D), lambda b, j: (b, 0, 0)),
            pl.BlockSpec((1, _NB, _D), lambda b, j: (b, j, 0)),
            pl.BlockSpec((1, _NB, _D), lambda b, j: (b, j, 0)),
        ],
        out_specs=[
            pl.BlockSpec((1, 1, _NB), lambda b, j: (b, 0, j)),
            pl.BlockSpec((1, 1, _NB), lambda b, j: (b, 0, j)),
        ],
        out_shape=[jax.ShapeDtypeStruct((_B, 1, _N), f32)] * 2,
        compiler_params=pltpu.CompilerParams(
            dimension_semantics=("parallel", "parallel")),
    )(v3, aud3, n_feats, n_auds)

    w = pl.pallas_call(
        _mask_body,
        in_specs=[
            pl.BlockSpec((1, 1), lambda: (0, 0)),
            pl.BlockSpec((_B, 1, _N), lambda: (0, 0, 0)),
            pl.BlockSpec((_B, 1, _N), lambda: (0, 0, 0)),
            pl.BlockSpec((_B, 1, _N), lambda: (0, 0, 0)),
        ],
        out_specs=pl.BlockSpec((_B, 1, _N), lambda: (0, 0, 0)),
        out_shape=jax.ShapeDtypeStruct((_B, 1, _N), f32),
        scratch_shapes=[pltpu.VMEM((_B, _N), f32), pltpu.VMEM((_B, _N), f32)],
    )(temp.reshape(1, 1), n_ids, s1, s2)

    aggr = pl.pallas_call(
        _aggr_body,
        grid=(_B, _NBLK),
        in_specs=[
            pl.BlockSpec((1, 1, _NB), lambda b, j: (b, 0, j)),
            pl.BlockSpec((1, _NB, _D), lambda b, j: (b, j, 0)),
        ],
        out_specs=pl.BlockSpec((1, 1, _D), lambda b, j: (b, 0, 0)),
        out_shape=jax.ShapeDtypeStruct((_B, 1, _D), f32),
        compiler_params=pltpu.CompilerParams(
            dimension_semantics=("parallel", "arbitrary")),
    )(w, n_answ)

    scores = pl.pallas_call(
        _final_body,
        in_specs=[
            pl.BlockSpec((_B, 1, _D), lambda: (0, 0, 0)),
            pl.BlockSpec((_B, 3, _D), lambda: (0, 0, 0)),
        ],
        out_specs=pl.BlockSpec((_B, 3), lambda: (0, 0)),
        out_shape=jax.ShapeDtypeStruct((_B, 3), f32),
    )(aggr, o)
    return scores


# speedup vs baseline: 3.4873x; 1.2360x over previous
import jax
import jax.numpy as jnp
from jax.experimental import pallas as pl
from jax.experimental.pallas import tpu as pltpu

_B, _N, _D = 32, 2048, 1024
_TOPK = 25
_NB = 1024
_NBLK = _N // _NB


def _score_body(v_ref, aud_ref, nf_ref, na_ref, s1_ref, s2_ref):
    x1 = nf_ref[0]
    x2 = na_ref[0]
    vb = v_ref[0]
    ab = aud_ref[0]
    d1 = jnp.sum(x1 * vb, axis=1).reshape(1, _NB)
    d2 = jnp.sum(x2 * ab, axis=1).reshape(1, _NB)
    ss1 = jnp.sum(x1 * x1, axis=1).reshape(1, _NB)
    ss2 = jnp.sum(x2 * x2, axis=1).reshape(1, _NB)
    nv = jnp.sqrt(jnp.sum(vb * vb))
    nu = jnp.sqrt(jnp.sum(ab * ab))
    n1 = jnp.sqrt(ss1)
    n2 = jnp.sqrt(ss2)
    nv_c = jnp.maximum(nv, 1e-12)
    nu_c = jnp.maximum(nu, 1e-12)
    n1_c = jnp.maximum(n1, 1e-12)
    n2_c = jnp.maximum(n2, 1e-12)
    c1 = (d1 / (nv_c * n1_c)) / jnp.maximum((nv / nv_c) * (n1 / n1_c), 1e-8)
    c2 = (d2 / (nu_c * n2_c)) / jnp.maximum((nu / nu_c) * (n2 / n2_c), 1e-8)
    s1_ref[0] = c1
    s2_ref[0] = c2


def _mask_body(temp_ref, ids_ref, s1_ref, s2_ref, w_ref, cur_ref, rem_ref):
    t = temp_ref[0, 0]
    a = 1.0 / (1.0 + jnp.exp(-t))
    mask = (ids_ref[:, 0, :] != -1).astype(jnp.float32)
    c1 = jnp.clip(s1_ref[:, 0, :] * mask, 0.0, 1.0)
    c2 = jnp.clip(s2_ref[:, 0, :] * mask, 0.0, 1.0)
    e1 = jnp.exp(c1 - jnp.max(c1, axis=-1, keepdims=True))
    sm1 = e1 / jnp.sum(e1, axis=-1, keepdims=True)
    e2 = jnp.exp(c2 - jnp.max(c2, axis=-1, keepdims=True))
    sm2 = e2 / jnp.sum(e2, axis=-1, keepdims=True)
    att = a * sm1 + (1.0 - a) * sm2
    iota = jax.lax.broadcasted_iota(jnp.int32, (_B, _N), 1)

    cur_ref[...] = att
    rem_ref[...] = jnp.zeros((_B, _N), jnp.float32)

    def step(_, c):
        cur = cur_ref[...]
        mx = jnp.max(cur, axis=-1, keepdims=True)
        cand = jnp.where(cur == mx, iota, _N)
        first = jnp.min(cand, axis=-1, keepdims=True)
        hit = iota == first
        cur_ref[...] = jnp.where(hit, -1.0, cur)
        rem_ref[...] = jnp.maximum(rem_ref[...], hit.astype(jnp.float32))
        return c

    jax.lax.fori_loop(0, _TOPK, step, 0)
    m = jnp.max(rem_ref[...], axis=0, keepdims=True)
    w_ref[:, 0, :] = att * m


def _aggr_body(w_ref, x_ref, out_ref):
    j = pl.program_id(1)
    part = jax.lax.dot_general(w_ref[0], x_ref[0], (((1,), (0,)), ((), ())),
                               preferred_element_type=jnp.float32)

    @pl.when(j == 0)
    def _():
        out_ref[0] = part

    @pl.when(j > 0)
    def _():
        out_ref[0] += part


def _final_body(aggr_ref, o_ref, out_ref):
    out_ref[...] = jnp.sum(aggr_ref[:, 0, :][:, None, :] * o_ref[...], axis=-1)


def kernel(v, aud, o, n_feats, n_ids, n_answ, n_auds, temp):
    f32 = jnp.float32
    v3 = v.reshape(_B, 1, _D)
    aud3 = aud.reshape(_B, 1, _D)
    s1, s2 = pl.pallas_call(
        _score_body,
        grid=(_B, _NBLK),
        in_specs=[
            pl.BlockSpec((1, 1, _D), lambda b, j: (b, 0, 0)),
            pl.BlockSpec((1, 1, _D), lambda b, j: (b, 0, 0)),
            pl.BlockSpec((1, _NB, _D), lambda b, j: (b, j, 0)),
            pl.BlockSpec((1, _NB, _D), lambda b, j: (b, j, 0)),
        ],
        out_specs=[
            pl.BlockSpec((1, 1, _NB), lambda b, j: (b, 0, j)),
            pl.BlockSpec((1, 1, _NB), lambda b, j: (b, 0, j)),
        ],
        out_shape=[jax.ShapeDtypeStruct((_B, 1, _N), f32)] * 2,
        compiler_params=pltpu.CompilerParams(
            dimension_semantics=("parallel", "parallel")),
    )(v3, aud3, n_feats, n_auds)

    w = pl.pallas_call(
        _mask_body,
        in_specs=[
            pl.BlockSpec((1, 1), lambda: (0, 0)),
            pl.BlockSpec((_B, 1, _N), lambda: (0, 0, 0)),
            pl.BlockSpec((_B, 1, _N), lambda: (0, 0, 0)),
            pl.BlockSpec((_B, 1, _N), lambda: (0, 0, 0)),
        ],
        out_specs=pl.BlockSpec((_B, 1, _N), lambda: (0, 0, 0)),
        out_shape=jax.ShapeDtypeStruct((_B, 1, _N), f32),
        scratch_shapes=[pltpu.VMEM((_B, _N), f32), pltpu.VMEM((_B, _N), f32)],
    )(temp.reshape(1, 1), n_ids, s1, s2)

    aggr = pl.pallas_call(
        _aggr_body,
        grid=(_B, _NBLK),
        in_specs=[
            pl.BlockSpec((1, 1, _NB), lambda b, j: (b, 0, j)),
            pl.BlockSpec((1, _NB, _D), lambda b, j: (b, j, 0)),
        ],
        out_specs=pl.BlockSpec((1, 1, _D), lambda b, j: (b, 0, 0)),
        out_shape=jax.ShapeDtypeStruct((_B, 1, _D), f32),
        compiler_params=pltpu.CompilerParams(
            dimension_semantics=("parallel", "arbitrary")),
    )(w, n_answ)

    scores = pl.pallas_call(
        _final_body,
        in_specs=[
            pl.BlockSpec((_B, 1, _D), lambda: (0, 0, 0)),
            pl.BlockSpec((_B, 3, _D), lambda: (0, 0, 0)),
        ],
        out_specs=pl.BlockSpec((_B, 3), lambda: (0, 0)),
        out_shape=jax.ShapeDtypeStruct((_B, 3), f32),
    )(aggr, o)
    return scores


# speedup vs baseline: 3.7823x; 1.0846x over previous
import jax
import jax.numpy as jnp
from jax.experimental import pallas as pl
from jax.experimental.pallas import tpu as pltpu

_B, _N, _D = 32, 2048, 1024
_TOPK = 25
_NB = 2048
_NBLK = _N // _NB


def _score_body(v_ref, aud_ref, nf_ref, na_ref, s1_ref, s2_ref):
    x1 = nf_ref[0]
    x2 = na_ref[0]
    vb = v_ref[0]
    ab = aud_ref[0]
    d1 = jnp.sum(x1 * vb, axis=1).reshape(1, _NB)
    d2 = jnp.sum(x2 * ab, axis=1).reshape(1, _NB)
    ss1 = jnp.sum(x1 * x1, axis=1).reshape(1, _NB)
    ss2 = jnp.sum(x2 * x2, axis=1).reshape(1, _NB)
    nv = jnp.sqrt(jnp.sum(vb * vb))
    nu = jnp.sqrt(jnp.sum(ab * ab))
    n1 = jnp.sqrt(ss1)
    n2 = jnp.sqrt(ss2)
    nv_c = jnp.maximum(nv, 1e-12)
    nu_c = jnp.maximum(nu, 1e-12)
    n1_c = jnp.maximum(n1, 1e-12)
    n2_c = jnp.maximum(n2, 1e-12)
    c1 = (d1 / (nv_c * n1_c)) / jnp.maximum((nv / nv_c) * (n1 / n1_c), 1e-8)
    c2 = (d2 / (nu_c * n2_c)) / jnp.maximum((nu / nu_c) * (n2 / n2_c), 1e-8)
    s1_ref[0] = c1
    s2_ref[0] = c2


def _mask_body(temp_ref, ids_ref, s1_ref, s2_ref, w_ref, cur_ref, rem_ref):
    t = temp_ref[0, 0]
    a = 1.0 / (1.0 + jnp.exp(-t))
    mask = (ids_ref[:, 0, :] != -1).astype(jnp.float32)
    c1 = jnp.clip(s1_ref[:, 0, :] * mask, 0.0, 1.0)
    c2 = jnp.clip(s2_ref[:, 0, :] * mask, 0.0, 1.0)
    e1 = jnp.exp(c1 - jnp.max(c1, axis=-1, keepdims=True))
    sm1 = e1 / jnp.sum(e1, axis=-1, keepdims=True)
    e2 = jnp.exp(c2 - jnp.max(c2, axis=-1, keepdims=True))
    sm2 = e2 / jnp.sum(e2, axis=-1, keepdims=True)
    att = a * sm1 + (1.0 - a) * sm2
    iota = jax.lax.broadcasted_iota(jnp.int32, (_B, _N), 1)

    cur_ref[...] = att
    rem_ref[...] = jnp.zeros((_B, _N), jnp.float32)

    def step(_, c):
        cur = cur_ref[...]
        mx = jnp.max(cur, axis=-1, keepdims=True)
        cand = jnp.where(cur == mx, iota, _N)
        first = jnp.min(cand, axis=-1, keepdims=True)
        hit = iota == first
        cur_ref[...] = jnp.where(hit, -1.0, cur)
        rem_ref[...] = jnp.maximum(rem_ref[...], hit.astype(jnp.float32))
        return c

    jax.lax.fori_loop(0, _TOPK, step, 0)
    m = jnp.max(rem_ref[...], axis=0, keepdims=True)
    w_ref[:, 0, :] = att * m


def _aggr_body(w_ref, x_ref, out_ref):
    j = pl.program_id(1)
    part = jax.lax.dot_general(w_ref[0], x_ref[0], (((1,), (0,)), ((), ())),
                               preferred_element_type=jnp.float32)

    @pl.when(j == 0)
    def _():
        out_ref[0] = part

    @pl.when(j > 0)
    def _():
        out_ref[0] += part


def _final_body(aggr_ref, o_ref, out_ref):
    out_ref[...] = jnp.sum(aggr_ref[:, 0, :][:, None, :] * o_ref[...], axis=-1)


def kernel(v, aud, o, n_feats, n_ids, n_answ, n_auds, temp):
    f32 = jnp.float32
    v3 = v.reshape(_B, 1, _D)
    aud3 = aud.reshape(_B, 1, _D)
    s1, s2 = pl.pallas_call(
        _score_body,
        grid=(_B, _NBLK),
        in_specs=[
            pl.BlockSpec((1, 1, _D), lambda b, j: (b, 0, 0)),
            pl.BlockSpec((1, 1, _D), lambda b, j: (b, 0, 0)),
            pl.BlockSpec((1, _NB, _D), lambda b, j: (b, j, 0)),
            pl.BlockSpec((1, _NB, _D), lambda b, j: (b, j, 0)),
        ],
        out_specs=[
            pl.BlockSpec((1, 1, _NB), lambda b, j: (b, 0, j)),
            pl.BlockSpec((1, 1, _NB), lambda b, j: (b, 0, j)),
        ],
        out_shape=[jax.ShapeDtypeStruct((_B, 1, _N), f32)] * 2,
        compiler_params=pltpu.CompilerParams(
            dimension_semantics=("parallel", "parallel")),
    )(v3, aud3, n_feats, n_auds)

    w = pl.pallas_call(
        _mask_body,
        in_specs=[
            pl.BlockSpec((1, 1), lambda: (0, 0)),
            pl.BlockSpec((_B, 1, _N), lambda: (0, 0, 0)),
            pl.BlockSpec((_B, 1, _N), lambda: (0, 0, 0)),
            pl.BlockSpec((_B, 1, _N), lambda: (0, 0, 0)),
        ],
        out_specs=pl.BlockSpec((_B, 1, _N), lambda: (0, 0, 0)),
        out_shape=jax.ShapeDtypeStruct((_B, 1, _N), f32),
        scratch_shapes=[pltpu.VMEM((_B, _N), f32), pltpu.VMEM((_B, _N), f32)],
    )(temp.reshape(1, 1), n_ids, s1, s2)

    aggr = pl.pallas_call(
        _aggr_body,
        grid=(_B, _NBLK),
        in_specs=[
            pl.BlockSpec((1, 1, _NB), lambda b, j: (b, 0, j)),
            pl.BlockSpec((1, _NB, _D), lambda b, j: (b, j, 0)),
        ],
        out_specs=pl.BlockSpec((1, 1, _D), lambda b, j: (b, 0, 0)),
        out_shape=jax.ShapeDtypeStruct((_B, 1, _D), f32),
        compiler_params=pltpu.CompilerParams(
            dimension_semantics=("parallel", "arbitrary")),
    )(w, n_answ)

    scores = pl.pallas_call(
        _final_body,
        in_specs=[
            pl.BlockSpec((_B, 1, _D), lambda: (0, 0, 0)),
            pl.BlockSpec((_B, 3, _D), lambda: (0, 0, 0)),
        ],
        out_specs=pl.BlockSpec((_B, 3), lambda: (0, 0)),
        out_shape=jax.ShapeDtypeStruct((_B, 3), f32),
    )(aggr, o)
    return scores


# speedup vs baseline: 4.1529x; 1.0980x over previous
import jax
import jax.numpy as jnp
from jax.experimental import pallas as pl
from jax.experimental.pallas import tpu as pltpu
from jax.experimental.pallas import tpu_sc as plsc

_B, _N, _D = 32, 2048, 1024
_TOPK = 25
_NB = 2048
_NBLK = _N // _NB
_CH = 32
_NCHUNK = _TOPK
_NE = _CH * _NCHUNK


def _score_body(v_ref, aud_ref, nf_ref, na_ref, s1_ref, s2_ref):
    x1 = nf_ref[0]
    x2 = na_ref[0]
    vb = v_ref[0]
    ab = aud_ref[0]
    d1 = jnp.sum(x1 * vb, axis=1).reshape(1, _NB)
    d2 = jnp.sum(x2 * ab, axis=1).reshape(1, _NB)
    ones = jnp.ones((1, _D), jnp.float32)
    dn = (((1,), (1,)), ((), ()))
    ss1 = jax.lax.dot_general(ones, x1 * x1, dn,
                              preferred_element_type=jnp.float32)
    ss2 = jax.lax.dot_general(ones, x2 * x2, dn,
                              preferred_element_type=jnp.float32)
    nv = jnp.sqrt(jnp.sum(vb * vb))
    nu = jnp.sqrt(jnp.sum(ab * ab))
    n1 = jnp.sqrt(ss1)
    n2 = jnp.sqrt(ss2)
    nv_c = jnp.maximum(nv, 1e-12)
    nu_c = jnp.maximum(nu, 1e-12)
    n1_c = jnp.maximum(n1, 1e-12)
    n2_c = jnp.maximum(n2, 1e-12)
    c1 = (d1 / (nv_c * n1_c)) / jnp.maximum((nv / nv_c) * (n1 / n1_c), 1e-8)
    c2 = (d2 / (nu_c * n2_c)) / jnp.maximum((nu / nu_c) * (n2 / n2_c), 1e-8)
    s1_ref[0] = c1
    s2_ref[0] = c2


def _mask_body(temp_ref, ids_ref, s1_ref, s2_ref, wl_ref, idx_ref,
               cur_ref, rem_ref, idxm_ref):
    t = temp_ref[0, 0]
    a = 1.0 / (1.0 + jnp.exp(-t))
    mask = (ids_ref[:, 0, :] != -1).astype(jnp.float32)
    c1 = jnp.clip(s1_ref[:, 0, :] * mask, 0.0, 1.0)
    c2 = jnp.clip(s2_ref[:, 0, :] * mask, 0.0, 1.0)
    e1 = jnp.exp(c1 - jnp.max(c1, axis=-1, keepdims=True))
    sm1 = e1 / jnp.sum(e1, axis=-1, keepdims=True)
    e2 = jnp.exp(c2 - jnp.max(c2, axis=-1, keepdims=True))
    sm2 = e2 / jnp.sum(e2, axis=-1, keepdims=True)
    att = a * sm1 + (1.0 - a) * sm2
    iota = jax.lax.broadcasted_iota(jnp.int32, (_B, _N), 1)
    iota32 = jax.lax.broadcasted_iota(jnp.int32, (_B, 32), 1)

    cur_ref[...] = att
    rem_ref[...] = jnp.zeros((_B, _N), jnp.float32)
    idxm_ref[...] = jnp.zeros((_B, 32), jnp.int32)

    def step(i, c):
        cur = cur_ref[...]
        mx = jnp.max(cur, axis=-1, keepdims=True)
        cand = jnp.where(cur == mx, iota, _N)
        first = jnp.min(cand, axis=-1, keepdims=True)
        hit = iota == first
        cur_ref[...] = jnp.where(hit, -1.0, cur)
        rem_ref[...] = jnp.maximum(rem_ref[...], hit.astype(jnp.float32))
        idxm_ref[...] = jnp.where(iota32 == i, first, idxm_ref[...])
        return c

    jax.lax.fori_loop(0, _TOPK, step, 0)
    rem = rem_ref[...]
    m = jnp.max(rem, axis=0, keepdims=True)
    cnt = jnp.sum(rem, axis=0, keepdims=True)
    w2 = att * m / jnp.maximum(cnt, 1.0)
    idx_t = idxm_ref[...].T
    iota_n = jax.lax.broadcasted_iota(jnp.int32, (_N, 32), 0)
    slabs = []
    for i in range(_NCHUNK):
        oh = (iota_n == idx_t[i:i + 1, :]).astype(jnp.float32)
        wl = jax.lax.dot_general(w2, oh, (((1,), (0,)), ((), ())),
                                 preferred_element_type=jnp.float32)
        slabs.append(wl)
    wl_ref[...] = jnp.concatenate(slabs, axis=1)
    idx_ref[...] = idx_t


def _sc_aggr(wl_hbm, idx_hbm, answ_hbm, out_hbm,
             idxv, wrow, wsp0, wsp1, rows0, rows1, acc_ref, sem0, sem1):
    cidx = jax.lax.axis_index("c")
    sidx = jax.lax.axis_index("s")
    b = sidx * 2 + cidx
    pltpu.sync_copy(idx_hbm, idxv)
    pltpu.sync_copy(wl_hbm.at[b], wrow)
    for k in range(_D // 16):
        acc_ref[pl.ds(k * 16, 16)] = jnp.zeros((16,), jnp.float32)

    ones16 = jnp.ones((16,), jnp.float32)

    def start(c, rows_ref, sem):
        pltpu.async_copy(answ_hbm.at[b].at[idxv.at[c]], rows_ref, sem)

    def wait(rows_ref, sem):
        pltpu.make_async_copy(answ_hbm.at[b].at[idxv.at[0]], rows_ref, sem).wait()

    def prep(c, wsp_ref):
        for h in range(_CH // 16):
            wvec = wrow[pl.ds(c * _CH + h * 16, 16)]
            for l in range(16):
                wsp_ref[h * 16 + l] = wvec[l] * ones16

    def fma(rows_ref, wsp_ref):
        for db in range(_D // 128):
            regs = tuple(acc_ref[pl.ds(db * 128 + a * 16, 16)] for a in range(8))

            def ebody(e, rg):
                spl = wsp_ref[e]
                return tuple(
                    rg[a] + spl * rows_ref[e, pl.ds(db * 128 + a * 16, 16)]
                    for a in range(8))

            regs = jax.lax.fori_loop(0, _CH, ebody, regs)
            for a in range(8):
                acc_ref[pl.ds(db * 128 + a * 16, 16)] = regs[a]

    start(0, rows0, sem0)

    def outer(g, carry):
        start(2 * g + 1, rows1, sem1)
        wait(rows0, sem0)
        prep(2 * g, wsp0)
        fma(rows0, wsp0)
        start(2 * g + 2, rows0, sem0)
        wait(rows1, sem1)
        prep(2 * g + 1, wsp1)
        fma(rows1, wsp1)
        return carry

    jax.lax.fori_loop(0, (_NCHUNK - 1) // 2, outer, 0)
    wait(rows0, sem0)
    prep(_NCHUNK - 1, wsp0)
    fma(rows0, wsp0)
    pltpu.sync_copy(acc_ref, out_hbm.at[b])


def _final_body(aggr_ref, o_ref, out_ref):
    out_ref[...] = jnp.sum(aggr_ref[:, 0, :][:, None, :] * o_ref[...], axis=-1)


def kernel(v, aud, o, n_feats, n_ids, n_answ, n_auds, temp):
    f32 = jnp.float32
    v3 = v.reshape(_B, 1, _D)
    aud3 = aud.reshape(_B, 1, _D)
    s1, s2 = pl.pallas_call(
        _score_body,
        grid=(_B, _NBLK),
        in_specs=[
            pl.BlockSpec((1, 1, _D), lambda b, j: (b, 0, 0)),
            pl.BlockSpec((1, 1, _D), lambda b, j: (b, 0, 0)),
            pl.BlockSpec((1, _NB, _D), lambda b, j: (b, j, 0)),
            pl.BlockSpec((1, _NB, _D), lambda b, j: (b, j, 0)),
        ],
        out_specs=[
            pl.BlockSpec((1, 1, _NB), lambda b, j: (b, 0, j)),
            pl.BlockSpec((1, 1, _NB), lambda b, j: (b, 0, j)),
        ],
        out_shape=[jax.ShapeDtypeStruct((_B, 1, _N), f32)] * 2,
        compiler_params=pltpu.CompilerParams(
            dimension_semantics=("parallel", "parallel")),
    )(v3, aud3, n_feats, n_auds)

    wlist, idx = pl.pallas_call(
        _mask_body,
        in_specs=[
            pl.BlockSpec((1, 1), lambda: (0, 0)),
            pl.BlockSpec((_B, 1, _N), lambda: (0, 0, 0)),
            pl.BlockSpec((_B, 1, _N), lambda: (0, 0, 0)),
            pl.BlockSpec((_B, 1, _N), lambda: (0, 0, 0)),
        ],
        out_specs=[
            pl.BlockSpec((_B, _NE), lambda: (0, 0)),
            pl.BlockSpec((32, 32), lambda: (0, 0)),
        ],
        out_shape=[
            jax.ShapeDtypeStruct((_B, _NE), f32),
            jax.ShapeDtypeStruct((32, 32), jnp.int32),
        ],
        scratch_shapes=[
            pltpu.VMEM((_B, _N), f32),
            pltpu.VMEM((_B, _N), f32),
            pltpu.VMEM((_B, 32), jnp.int32),
        ],
    )(temp.reshape(1, 1), n_ids, s1, s2)

    aggr = pl.kernel(
        _sc_aggr,
        out_type=jax.ShapeDtypeStruct((_B, _D), f32),
        mesh=plsc.VectorSubcoreMesh(core_axis_name="c", subcore_axis_name="s"),
        scratch_types=[
            pltpu.VMEM((32, 32), jnp.int32),
            pltpu.VMEM((_NE,), f32),
            pltpu.VMEM((_CH, 16), f32),
            pltpu.VMEM((_CH, 16), f32),
            pltpu.VMEM((_CH, _D), f32),
            pltpu.VMEM((_CH, _D), f32),
            pltpu.VMEM((_D,), f32),
            pltpu.SemaphoreType.DMA,
            pltpu.SemaphoreType.DMA,
        ],
    )(wlist, idx, n_answ)

    scores = pl.pallas_call(
        _final_body,
        in_specs=[
            pl.BlockSpec((_B, 1, _D), lambda: (0, 0, 0)),
            pl.BlockSpec((_B, 3, _D), lambda: (0, 0, 0)),
        ],
        out_specs=pl.BlockSpec((_B, 3), lambda: (0, 0)),
        out_shape=jax.ShapeDtypeStruct((_B, 3), f32),
    )(aggr.reshape(_B, 1, _D), o)
    return scores
